# Initial kernel scaffold; baseline (speedup 1.0000x reference)
import jax
import jax.numpy as jnp
from jax.experimental import pallas as pl


def kernel(points_xyz, points_feature, points_mask, params):
    raise NotImplementedError("write your pallas kernel here")


# speedup vs baseline: 1.3311x; 1.3311x over previous
import functools
import jax
import jax.numpy as jnp
from jax.experimental import pallas as pl

XY_NX, XY_NY = 432, 496
CYL_NX, CYL_NY = 2560, 100
XY_RX = (0.0, 69.12); XY_RY = (-39.68, 39.68); XY_RZ = (-3.0, 1.0)
PI = 3.14159265358979323846
CYL_RX = (-PI, PI); CYL_RY = (-3.0, 1.0); CYL_RZ = (0.0, 69.12)


def _view_prep(px, py, pz, m, nx, ny, rx, ry, rz):
    sx = (rx[1] - rx[0]) / nx
    sy = (ry[1] - ry[0]) / ny
    sz = (rz[1] - rz[0]) / 1.0
    fx = (px - rx[0]) / sx
    fy = (py - ry[0]) / sy
    fz = (pz - rz[0]) / sz
    vx = jnp.floor(fx); vy = jnp.floor(fy); vz = jnp.floor(fz)
    inr = ((vx >= 0) & (vx < nx) & (vy >= 0) & (vy < ny)
           & (vz >= 0) & (vz < 1))
    vxc = jnp.clip(vx, 0.0, nx - 1.0)
    vyc = jnp.clip(vy, 0.0, ny - 1.0)
    vzc = jnp.clip(vz, 0.0, 0.0)
    idx = vxc.astype(jnp.int32) * ny + vyc.astype(jnp.int32)
    valid = (inr & (m > 0)).astype(jnp.float32)
    cx = rx[0] + (vxc + 0.5) * sx
    cy = ry[0] + (vyc + 0.5) * sy
    cz = rz[0] + (vzc + 0.5) * sz
    return idx, valid, cx, cy, cz, fx, fy


def _prep_kernel(x_ref, y_ref, z_ref, m_ref, *outs):
    x = x_ref[...]; y = y_ref[...]; z = z_ref[...]; m = m_ref[...]
    oxy = _view_prep(x, y, z, m, XY_NX, XY_NY, XY_RX, XY_RY, XY_RZ)
    pa = jnp.arctan2(y, x)
    pr = jnp.sqrt(x * x + y * y)
    ocyl = _view_prep(pa, z, pr, m, CYL_NX, CYL_NY, CYL_RX, CYL_RY, CYL_RZ)
    vals = list(oxy) + [pa, pr] + list(ocyl)
    for r, v in zip(outs, vals):
        r[...] = v


def _prep(x, y, z, m):
    B, N = x.shape
    f32 = jnp.float32
    shapes = ([jax.ShapeDtypeStruct((B, N), jnp.int32)]
              + [jax.ShapeDtypeStruct((B, N), f32)] * 6
              + [jax.ShapeDtypeStruct((B, N), f32)] * 2
              + [jax.ShapeDtypeStruct((B, N), jnp.int32)]
              + [jax.ShapeDtypeStruct((B, N), f32)] * 6)
    return pl.pallas_call(_prep_kernel, out_shape=shapes)(x, y, z, m)


def _mm_stats(x, w, b=None, block=2048):
    R, K = x.shape
    C = w.shape[1]
    assert R % block == 0, (R, block)
    g = R // block

    def kfn(x_ref, w_ref, b_ref, h_ref, s_ref, s2_ref):
        h = jnp.dot(x_ref[...], w_ref[...],
                    preferred_element_type=jnp.float32) + b_ref[...]
        h_ref[...] = h

        @pl.when(pl.program_id(0) == 0)
        def _():
            s_ref[...] = jnp.zeros_like(s_ref)
            s2_ref[...] = jnp.zeros_like(s2_ref)

        s_ref[...] += jnp.sum(h, axis=0, keepdims=True)
        s2_ref[...] += jnp.sum(h * h, axis=0, keepdims=True)

    bb = b.reshape(1, C) if b is not None else jnp.zeros((1, C), jnp.float32)
    h, s, s2 = pl.pallas_call(
        kfn,
        grid=(g,),
        in_specs=[pl.BlockSpec((block, K), lambda i: (i, 0)),
                  pl.BlockSpec((K, C), lambda i: (0, 0)),
                  pl.BlockSpec((1, C), lambda i: (0, 0))],
        out_specs=[pl.BlockSpec((block, C), lambda i: (i, 0)),
                   pl.BlockSpec((1, C), lambda i: (0, 0)),
                   pl.BlockSpec((1, C), lambda i: (0, 0))],
        out_shape=[jax.ShapeDtypeStruct((R, C), jnp.float32),
                   jax.ShapeDtypeStruct((1, C), jnp.float32),
                   jax.ShapeDtypeStruct((1, C), jnp.float32)],
    )(x, w, bb)
    return h, s, s2


def _sumsq_dev(h, mean, block=2048):
    R, C = h.shape
    g = R // block

    def kfn(h_ref, m_ref, s_ref):
        @pl.when(pl.program_id(0) == 0)
        def _():
            s_ref[...] = jnp.zeros_like(s_ref)
        d = h_ref[...] - m_ref[...]
        s_ref[...] += jnp.sum(d * d, axis=0, keepdims=True)

    return pl.pallas_call(
        kfn,
        grid=(g,),
        in_specs=[pl.BlockSpec((block, C), lambda i: (i, 0)),
                  pl.BlockSpec((1, C), lambda i: (0, 0))],
        out_specs=pl.BlockSpec((1, C), lambda i: (0, 0)),
        out_shape=jax.ShapeDtypeStruct((1, C), jnp.float32),
    )(h, mean)


def _affine(h, scale, shift, act, mask=None, block=2048):
    R, C = h.shape
    g = R // block

    def kfn(*refs):
        if mask is not None:
            h_ref, sc_ref, sh_ref, mk_ref, o_ref = refs
        else:
            h_ref, sc_ref, sh_ref, o_ref = refs
        v = h_ref[...] * sc_ref[...] + sh_ref[...]
        if act == 'relu':
            v = jnp.maximum(v, 0.0)
        elif act == 'sigmoid':
            v = 1.0 / (1.0 + jnp.exp(-v))
        if mask is not None:
            v = v * mk_ref[...]
        o_ref[...] = v

    in_specs = [pl.BlockSpec((block, C), lambda i: (i, 0)),
                pl.BlockSpec((1, C), lambda i: (0, 0)),
                pl.BlockSpec((1, C), lambda i: (0, 0))]
    args = [h, scale, shift]
    if mask is not None:
        in_specs.append(pl.BlockSpec((block, 1), lambda i: (i, 0)))
        args.append(mask)
    return pl.pallas_call(
        kfn,
        grid=(g,),
        in_specs=in_specs,
        out_specs=pl.BlockSpec((block, C), lambda i: (i, 0)),
        out_shape=jax.ShapeDtypeStruct((R, C), jnp.float32),
    )(*args)


def _gate(fw, bw, front, bev, block=2048):
    R, C = fw.shape
    g = R // block

    def kfn(a_ref, b_ref, f_ref, v_ref, o_ref):
        o_ref[...] = a_ref[...] * f_ref[...] + b_ref[...] * v_ref[...]

    spec = pl.BlockSpec((block, C), lambda i: (i, 0))
    return pl.pallas_call(
        kfn,
        grid=(g,),
        in_specs=[spec] * 4,
        out_specs=spec,
        out_shape=jax.ShapeDtypeStruct((R, C), jnp.float32),
    )(fw, bw, front, bev)


def _bn_scale_shift(g, b, mean, var, eps):
    scale = (g / jnp.sqrt(var + eps)).reshape(1, -1)
    shift = b.reshape(1, -1) - mean * scale
    return scale, shift


def _stats(p3, idx, w, V):
    B, N, _ = p3.shape
    seg = lambda d, i: jax.ops.segment_sum(d, i, num_segments=V)
    c = jax.vmap(seg)(w, idx)
    s = jax.vmap(seg)(p3 * w[..., None], idx)
    mean = s / jnp.maximum(c, 1.0)[..., None]
    centroids = jnp.take_along_axis(mean, idx[..., None], axis=1)
    cen = (p3 - centroids) * w[..., None]
    outer = (cen[..., :, None] * cen[..., None, :]).reshape(B, N, 9)
    covs = jax.vmap(seg)(outer, idx) / jnp.maximum(c, 1.0)[..., None]
    covpt = jnp.take_along_axis(covs, idx[..., None], axis=1)
    vpc = jnp.take_along_axis(c, idx, axis=1)
    return p3 - centroids, covpt, centroids, vpc


def _seg_max(data, idx, V):
    out = jax.vmap(lambda dd, ii: jax.ops.segment_max(dd, ii, num_segments=V))(
        data, idx)
    return jnp.where(out < -1e8, 0.0, out)


def _pointnet(f, w, g, b, eps, maskr, two_pass=True):
    R = f.shape[0]
    h, s, s2 = _mm_stats(f, w)
    mean = s / R
    if two_pass:
        var = _sumsq_dev(h, mean) / R
    else:
        var = s2 / R - mean * mean
    scale, shift = _bn_scale_shift(g, b, mean, var, eps)
    return _affine(h, scale, shift, 'relu', mask=maskr)


def _view_branch(f23, maskr, idx, valid, P, pre, V, gx, gy, fx, fy,
                 conv_block):
    B, N = idx.shape
    R = B * N
    hact = _pointnet(f23, P[pre + 'pn_w'], P[pre + 'pn_g'], P[pre + 'pn_b'],
                     1e-5, maskr)
    d = jnp.where(valid.reshape(R, 1) > 0, hact, -1e9).reshape(B, N, 64)
    grid = _seg_max(d, idx, V)
    rows = grid.reshape(B * V, 64)
    y, ys, ys2 = _mm_stats(rows, P[pre + 'conv_w'], block=conv_block)
    cnt = B * V
    meanY = ys / cnt
    varY = ys2 / cnt - meanY * meanY
    scale, shift = _bn_scale_shift(P[pre + 'conv_g'], P[pre + 'conv_b'],
                                   meanY, varY, 1e-3)
    img = _affine(y, scale, shift, 'relu', block=conv_block).reshape(B, V, 64)
    xq = jnp.clip(fx, 0.0, gx - 1.0)
    yq = jnp.clip(fy, 0.0, gy - 1.0)
    x0 = jnp.floor(xq); y0 = jnp.floor(yq)
    x0i = x0.astype(jnp.int32); y0i = y0.astype(jnp.int32)
    x1i = jnp.minimum(x0i + 1, gx - 1); y1i = jnp.minimum(y0i + 1, gy - 1)
    take = lambda r: jnp.take_along_axis(img, r[..., None], axis=1)
    Ia = take(x0i * gy + y0i); Ib = take(x1i * gy + y0i)
    Ic = take(x0i * gy + y1i); Id = take(x1i * gy + y1i)
    xd = (xq - x0)[..., None]; yd = (yq - y0)[..., None]
    out = (Ia * (1 - xd) * (1 - yd) + Ib * xd * (1 - yd)
           + Ic * (1 - xd) * yd + Id * xd * yd)
    return out.reshape(R, 64)


def _mlp(xc, P, pre):
    R = xc.shape[0]
    h1, s, s2 = _mm_stats(xc, P[pre + 'w1'], P[pre + 'b1'])
    m1 = s / R
    v1 = s2 / R - m1 * m1
    sc1, sh1 = _bn_scale_shift(P[pre + 'g1'], P[pre + 'be1'], m1, v1, 1e-3)
    a1 = _affine(h1, sc1, sh1, 'relu')
    h2, s, s2 = _mm_stats(a1, P[pre + 'w2'], P[pre + 'b2'])
    m2 = s / R
    v2 = s2 / R - m2 * m2
    sc2, sh2 = _bn_scale_shift(P[pre + 'g2'], P[pre + 'be2'], m2, v2, 1e-3)
    return _affine(h2, sc2, sh2, 'sigmoid')


def kernel(points_xyz, points_feature, points_mask, params):
    P = params
    B, N, _ = points_xyz.shape
    R = B * N
    VXY = XY_NX * XY_NY
    VCYL = CYL_NX * CYL_NY
    x = points_xyz[..., 0]; y = points_xyz[..., 1]; z = points_xyz[..., 2]
    (idxxy, valxy, cx, cy, cz, fxx, fxy,
     pa, pr,
     idxc, valc, ccx, ccy, ccz, fcx, fcy) = _prep(x, y, z, points_mask)

    p_xy = points_xyz
    p_cyl = jnp.stack([pa, z, pr], axis=-1)
    cen_xy = jnp.stack([cx, cy, cz], axis=-1)
    cen_cyl = jnp.stack([ccx, ccy, ccz], axis=-1)

    cxyz_xy, cov_xy, ctr_xy, vpc_xy = _stats(p_xy, idxxy, valxy, VXY)
    cxyz_c, cov_c, ctr_c, vpc_c = _stats(p_cyl, idxc, valc, VCYL)

    feat = points_feature[..., None]
    f_xy = jnp.concatenate([p_xy, p_xy - cen_xy, vpc_xy[..., None],
                            cxyz_xy, cov_xy, ctr_xy, feat], axis=-1)
    f_cyl = jnp.concatenate([p_cyl, p_cyl - cen_cyl, vpc_c[..., None],
                             cxyz_c, cov_c, ctr_c, feat], axis=-1)

    maskr = points_mask.reshape(R, 1)
    bev = _view_branch(f_xy.reshape(R, 23), maskr, idxxy, valxy, P, 'xy_',
                       VXY, XY_NX, XY_NY, fxx, fxy, conv_block=1536)
    front = _view_branch(f_cyl.reshape(R, 23), maskr, idxc, valc, P, 'cyl_',
                         VCYL, CYL_NX, CYL_NY, fcx, fcy, conv_block=2048)

    xc = jnp.concatenate([bev, front], axis=-1)
    fw = _mlp(xc, P, 'mlp1_')
    bw = _mlp(xc, P, 'mlp2_')
    gated = _gate(fw, bw, front, bev)
    x192 = jnp.concatenate([gated, bev, front], axis=-1)

    xf = _pointnet(x192, P['pn_w'], P['pn_g'], P['pn_b'], 1e-5, maskr,
                   two_pass=False)
    d = jnp.where(valxy.reshape(R, 1) > 0, xf, -1e9).reshape(B, N, 128)
    pil = _seg_max(d, idxxy, VXY)
    return jnp.transpose(pil.reshape(B, XY_NX, XY_NY, 128), (0, 3, 2, 1))
